# baseline (speedup 1.0000x reference)
import functools

import jax
import jax.numpy as jnp
from jax import lax
from jax.experimental import pallas as pl
from jax.experimental.pallas import tpu as pltpu
from jax.experimental.pallas import tpu_sc as plsc

_NUM_FIELDS = 26
_VOCAB = 100000
_EMB = 32
_B = 4096
_NF = 33
_NC = 2
_NS = 16
_NW = _NC * _NS
_RPW = _B // _NW
_CHUNK = 64
_NCH = _RPW // _CHUNK
_HALF = 16


_TBL = tuple(range(1, 26)) + tuple(range(1, 9))


def _sc_body(g_hbm, t_hbm, w_hbm, bias_hbm, out_hbm,
             idx_v, buf_v, acc_v, outc_v, wv_v, bias_v, sem):
    wid = lax.axis_index("s") * _NC + lax.axis_index("c")
    pltpu.sync_copy(w_hbm, wv_v)
    pltpu.sync_copy(bias_hbm, bias_v)

    for c in range(_NCH):
        pltpu.sync_copy(g_hbm.at[wid, c], idx_v)

        for j, tbl in enumerate(_TBL):
            pltpu.async_copy(t_hbm.at[tbl].at[idx_v.at[j]],
                             buf_v.at[pl.ds(j * _CHUNK, _CHUNK)], sem)
        pltpu.make_async_copy(t_hbm.at[0].at[pl.ds(0, _NF * _CHUNK)],
                              buf_v, sem).wait()

        for j in range(_NF):
            w_lo = wv_v[j, pl.ds(0, _HALF)]
            w_hi = wv_v[j, pl.ds(_HALF, _HALF)]

            def row_body(r, carry, j=j, w_lo=w_lo, w_hi=w_hi):
                lo = buf_v[j * _CHUNK + r, pl.ds(0, _HALF)] * w_lo
                hi = buf_v[j * _CHUNK + r, pl.ds(_HALF, _HALF)] * w_hi
                if j == 0:
                    acc_v[pl.ds(r * _EMB, _HALF)] = lo
                    acc_v[pl.ds(r * _EMB + _HALF, _HALF)] = hi
                else:
                    plsc.addupdate(acc_v.at[pl.ds(r * _EMB, _HALF)], lo)
                    plsc.addupdate(acc_v.at[pl.ds(r * _EMB + _HALF, _HALF)], hi)
                return carry

            lax.fori_loop(0, _CHUNK, row_body, 0)

        bias = bias_v[:]
        for r16 in range(_CHUNK // _HALF):
            ridx = (lax.iota(jnp.int32, _HALF) + r16 * _HALF) * _EMB
            tot = plsc.load_gather(acc_v, [ridx])
            for k in range(1, _EMB):
                tot = tot + plsc.load_gather(acc_v, [ridx + k])
            tot = tot + bias
            p = 1.0 / (1.0 + jnp.exp(-tot))
            outc_v[pl.ds(c * _CHUNK + r16 * _HALF, _HALF)] = p

    pltpu.sync_copy(outc_v, out_hbm.at[pl.ds(wid * _RPW, _RPW)])


@functools.partial(jax.jit, static_argnums=())
def kernel(ad_feature_inputs, feature_inputs, tables, W_meta, b_meta,
           W_pred, b_pred):
    w0 = W_pred[0, :_EMB]
    v = W_meta.T @ w0
    w_eff = jnp.concatenate(
        [W_pred[0, _EMB:].reshape(_NF - 8, _EMB),
         jnp.broadcast_to((v / _EMB)[:, None], (8, _EMB))], axis=0)
    bias_c = b_pred[0] + jnp.dot(b_meta, w0)
    bias_vec = jnp.full((_HALF,), bias_c, jnp.float32)

    g = jnp.concatenate([feature_inputs, ad_feature_inputs], axis=1)
    g = g.reshape(_NW, _NCH, _CHUNK, _NF).transpose(0, 1, 3, 2)

    mesh = plsc.VectorSubcoreMesh(core_axis_name="c", subcore_axis_name="s")
    out = pl.kernel(
        _sc_body,
        out_type=jax.ShapeDtypeStruct((_B,), jnp.float32),
        mesh=mesh,
        compiler_params=pltpu.CompilerParams(needs_layout_passes=False,
                                             use_tc_tiling_on_sc=False),
        scratch_types=[
            pltpu.VMEM((_NF, _CHUNK), jnp.int32),
            pltpu.VMEM((_NF * _CHUNK, _EMB), jnp.float32),
            pltpu.VMEM((_CHUNK * _EMB,), jnp.float32),
            pltpu.VMEM((_RPW,), jnp.float32),
            pltpu.VMEM((_NF, _EMB), jnp.float32),
            pltpu.VMEM((_HALF,), jnp.float32),
            pltpu.SemaphoreType.DMA,
        ],
    )(g, tables, w_eff, bias_vec)
    return out[:, None]


# speedup vs baseline: 2.7295x; 2.7295x over previous
import functools

import jax
import jax.numpy as jnp
from jax import lax
from jax.experimental import pallas as pl
from jax.experimental.pallas import tpu as pltpu
from jax.experimental.pallas import tpu_sc as plsc

_NUM_FIELDS = 26
_VOCAB = 100000
_VPAD = 102400
_EMB = 32
_B = 4096
_NF = 33
_NT = 25
_NC = 2
_NS = 16
_NW = _NC * _NS
_RPW = _B // _NW
_HALF = 16
_VB = 4096
_NVB = _VPAD // _VB


def _score_body(w_ref, t_ref, of_ref, oa_ref):
    s = lax.dot_general(w_ref[0], t_ref[0], (((1,), (0,)), ((), ())),
                        preferred_element_type=jnp.float32)
    of_ref[...] = s[0]
    oa_ref[...] = s[1]


def _sc_body(g_hbm, of_hbm, oa_hbm, bias_hbm, out_hbm,
             gidx_v, sbuf_v, outv_v, bias_v, sem):
    wid = lax.axis_index("s") * _NC + lax.axis_index("c")
    pltpu.sync_copy(bias_hbm, bias_v)
    pltpu.sync_copy(g_hbm.at[wid], gidx_v)
    for j in range(_NF):
        src = of_hbm if j < _NT else oa_hbm
        pltpu.async_copy(src.at[gidx_v.at[j]],
                         sbuf_v.at[pl.ds(j * _RPW, _RPW)], sem)
    pltpu.make_async_copy(of_hbm.at[pl.ds(0, _NF * _RPW)], sbuf_v, sem).wait()

    bias = bias_v[:]
    for g in range(_RPW // _HALF):
        tot = sbuf_v[pl.ds(g * _HALF, _HALF)]
        for j in range(1, _NF):
            tot = tot + sbuf_v[pl.ds(j * _RPW + g * _HALF, _HALF)]
        tot = tot + bias
        p = 1.0 / (1.0 + jnp.exp(-tot))
        outv_v[pl.ds(g * _HALF, _HALF)] = p
    pltpu.sync_copy(outv_v, out_hbm.at[pl.ds(wid * _RPW, _RPW)])


def kernel(ad_feature_inputs, feature_inputs, tables, W_meta, b_meta,
           W_pred, b_pred):
    t_t = tables.transpose(0, 2, 1)

    w0 = W_pred[0, :_EMB]
    v = W_meta.T @ w0
    w_feat = W_pred[0, _EMB:].reshape(_NT, _EMB)
    w_ad = jnp.zeros((_NT, _EMB), jnp.float32).at[:8].set(
        jnp.broadcast_to((v / _EMB)[:, None], (8, _EMB)))
    w_all = jnp.stack([w_feat, w_ad], axis=1)
    bias_c = b_pred[0] + jnp.dot(b_meta, w0)
    bias_vec = jnp.full((_HALF,), bias_c, jnp.float32)

    o_feat, o_ad = pl.pallas_call(
        _score_body,
        grid=(_NT, _NVB),
        in_specs=[
            pl.BlockSpec((1, 2, _EMB), lambda t, vb: (t, 0, 0)),
            pl.BlockSpec((1, _EMB, _VB), lambda t, vb: (t + 1, 0, vb)),
        ],
        out_specs=[
            pl.BlockSpec((_VB,), lambda t, vb: (t * _NVB + vb)),
            pl.BlockSpec((_VB,), lambda t, vb: (t * _NVB + vb)),
        ],
        out_shape=[
            jax.ShapeDtypeStruct((_NT * _VPAD,), jnp.float32),
            jax.ShapeDtypeStruct((_NT * _VPAD,), jnp.float32),
        ],
    )(w_all, t_t)

    offs = jnp.arange(_NT, dtype=jnp.int32) * _VPAD
    g = jnp.concatenate([feature_inputs + offs[None, :],
                         ad_feature_inputs + offs[None, :8]], axis=1)
    g = g.reshape(_NW, _RPW, _NF).transpose(0, 2, 1)

    mesh = plsc.VectorSubcoreMesh(core_axis_name="c", subcore_axis_name="s")
    out = pl.kernel(
        _sc_body,
        out_type=jax.ShapeDtypeStruct((_B,), jnp.float32),
        mesh=mesh,
        compiler_params=pltpu.CompilerParams(needs_layout_passes=False,
                                             use_tc_tiling_on_sc=False),
        scratch_types=[
            pltpu.VMEM((_NF, _RPW), jnp.int32),
            pltpu.VMEM((_NF * _RPW,), jnp.float32),
            pltpu.VMEM((_RPW,), jnp.float32),
            pltpu.VMEM((_HALF,), jnp.float32),
            pltpu.SemaphoreType.DMA,
        ],
    )(g, o_feat, o_ad, bias_vec)
    return out[:, None]


# speedup vs baseline: 4.2583x; 1.5601x over previous
import functools

import jax
import jax.numpy as jnp
from jax import lax
from jax.experimental import pallas as pl
from jax.experimental.pallas import tpu as pltpu
from jax.experimental.pallas import tpu_sc as plsc

_NUM_FIELDS = 26
_VOCAB = 100000
_VPAD = 106496
_EMB = 32
_B = 4096
_NF = 33
_NT = 25
_NC = 2
_NS = 16
_NW = _NC * _NS
_RPW = _B // _NW
_HALF = 16
_VB = 8192
_NVB = _VPAD // _VB


def _score_body(w_ref, t_ref, of_ref, oa_ref):
    s = lax.dot_general(w_ref[0], t_ref[0], (((1,), (0,)), ((), ())),
                        preferred_element_type=jnp.float32)
    of_ref[...] = s[0]
    oa_ref[...] = s[1]


def _sc_body(g_hbm, of_hbm, oa_hbm, bias_hbm, out_hbm,
             gidx_v, sbuf_v, outv_v, bias_v, sem):
    wid = lax.axis_index("s") * _NC + lax.axis_index("c")
    pltpu.sync_copy(bias_hbm, bias_v)
    pltpu.sync_copy(g_hbm.at[wid], gidx_v)
    for j in range(_NF):
        src = of_hbm if j < _NT else oa_hbm
        pltpu.async_copy(src.at[gidx_v.at[j]],
                         sbuf_v.at[pl.ds(j * _RPW, _RPW)], sem)
    pltpu.make_async_copy(of_hbm.at[pl.ds(0, _NF * _RPW)], sbuf_v, sem).wait()

    bias = bias_v[:]
    for g in range(_RPW // _HALF):
        tot = sbuf_v[pl.ds(g * _HALF, _HALF)]
        for j in range(1, _NF):
            tot = tot + sbuf_v[pl.ds(j * _RPW + g * _HALF, _HALF)]
        tot = tot + bias
        p = 1.0 / (1.0 + jnp.exp(-tot))
        outv_v[pl.ds(g * _HALF, _HALF)] = p
    pltpu.sync_copy(outv_v, out_hbm.at[pl.ds(wid * _RPW, _RPW)])


def kernel(ad_feature_inputs, feature_inputs, tables, W_meta, b_meta,
           W_pred, b_pred):
    t_t = tables.transpose(0, 2, 1)

    w0 = W_pred[0, :_EMB]
    v = W_meta.T @ w0
    w_feat = W_pred[0, _EMB:].reshape(_NT, _EMB)
    w_ad = jnp.zeros((_NT, _EMB), jnp.float32).at[:8].set(
        jnp.broadcast_to((v / _EMB)[:, None], (8, _EMB)))
    w_all = jnp.stack([w_feat, w_ad], axis=1)
    bias_c = b_pred[0] + jnp.dot(b_meta, w0)
    bias_vec = jnp.full((_HALF,), bias_c, jnp.float32)

    o_feat, o_ad = pl.pallas_call(
        _score_body,
        grid=(_NT, _NVB),
        in_specs=[
            pl.BlockSpec((1, 2, _EMB), lambda t, vb: (t, 0, 0)),
            pl.BlockSpec((1, _EMB, _VB), lambda t, vb: (t + 1, 0, vb)),
        ],
        out_specs=[
            pl.BlockSpec((_VB,), lambda t, vb: (t * _NVB + vb)),
            pl.BlockSpec((_VB,), lambda t, vb: (t * _NVB + vb)),
        ],
        out_shape=[
            jax.ShapeDtypeStruct((_NT * _VPAD,), jnp.float32),
            jax.ShapeDtypeStruct((_NT * _VPAD,), jnp.float32),
        ],
    )(w_all, t_t)

    offs = jnp.arange(_NT, dtype=jnp.int32) * _VPAD
    g = jnp.concatenate([feature_inputs + offs[None, :],
                         ad_feature_inputs + offs[None, :8]], axis=1)
    g = g.reshape(_NW, _RPW, _NF).transpose(0, 2, 1)

    mesh = plsc.VectorSubcoreMesh(core_axis_name="c", subcore_axis_name="s")
    out = pl.kernel(
        _sc_body,
        out_type=jax.ShapeDtypeStruct((_B,), jnp.float32),
        mesh=mesh,
        compiler_params=pltpu.CompilerParams(needs_layout_passes=False,
                                             use_tc_tiling_on_sc=False),
        scratch_types=[
            pltpu.VMEM((_NF, _RPW), jnp.int32),
            pltpu.VMEM((_NF * _RPW,), jnp.float32),
            pltpu.VMEM((_RPW,), jnp.float32),
            pltpu.VMEM((_HALF,), jnp.float32),
            pltpu.SemaphoreType.DMA,
        ],
    )(g, o_feat, o_ad, bias_vec)
    return out[:, None]


# speedup vs baseline: 5.8766x; 1.3800x over previous
import functools

import jax
import jax.numpy as jnp
from jax import lax
from jax.experimental import pallas as pl
from jax.experimental.pallas import tpu as pltpu
from jax.experimental.pallas import tpu_sc as plsc

_NUM_FIELDS = 26
_VOCAB = 100000
_VPAD = 114688
_EMB = 32
_B = 4096
_NF = 33
_NT = 25
_NC = 2
_NS = 16
_NW = _NC * _NS
_RPW = _B // _NW
_HALF = 16
_VB = 16384
_NVB = _VPAD // _VB


def _score_body(w_ref, t_ref, of_ref, oa_ref):
    s = lax.dot_general(w_ref[0], t_ref[0], (((1,), (0,)), ((), ())),
                        preferred_element_type=jnp.float32)
    of_ref[...] = s[0]
    oa_ref[...] = s[1]


def _sc_body(g_hbm, of_hbm, oa_hbm, bias_hbm, out_hbm,
             gidx_v, sbuf_v, outv_v, bias_v, sem):
    wid = lax.axis_index("s") * _NC + lax.axis_index("c")
    pltpu.sync_copy(bias_hbm, bias_v)
    pltpu.sync_copy(g_hbm.at[wid], gidx_v)
    for j in range(_NF):
        src = of_hbm if j < _NT else oa_hbm
        pltpu.async_copy(src.at[gidx_v.at[j]],
                         sbuf_v.at[pl.ds(j * _RPW, _RPW)], sem)
    pltpu.make_async_copy(of_hbm.at[pl.ds(0, _NF * _RPW)], sbuf_v, sem).wait()

    bias = bias_v[:]
    for g in range(_RPW // _HALF):
        tot = sbuf_v[pl.ds(g * _HALF, _HALF)]
        for j in range(1, _NF):
            tot = tot + sbuf_v[pl.ds(j * _RPW + g * _HALF, _HALF)]
        tot = tot + bias
        p = 1.0 / (1.0 + jnp.exp(-tot))
        outv_v[pl.ds(g * _HALF, _HALF)] = p
    pltpu.sync_copy(outv_v, out_hbm.at[pl.ds(wid * _RPW, _RPW)])


def kernel(ad_feature_inputs, feature_inputs, tables, W_meta, b_meta,
           W_pred, b_pred):
    t_t = tables.transpose(0, 2, 1)

    w0 = W_pred[0, :_EMB]
    v = W_meta.T @ w0
    w_feat = W_pred[0, _EMB:].reshape(_NT, _EMB)
    w_ad = jnp.zeros((_NT, _EMB), jnp.float32).at[:8].set(
        jnp.broadcast_to((v / _EMB)[:, None], (8, _EMB)))
    w_all = jnp.stack([w_feat, w_ad], axis=1)
    bias_c = b_pred[0] + jnp.dot(b_meta, w0)
    bias_vec = jnp.full((_HALF,), bias_c, jnp.float32)

    o_feat, o_ad = pl.pallas_call(
        _score_body,
        grid=(_NT, _NVB),
        in_specs=[
            pl.BlockSpec((1, 2, _EMB), lambda t, vb: (t, 0, 0)),
            pl.BlockSpec((1, _EMB, _VB), lambda t, vb: (t + 1, 0, vb)),
        ],
        out_specs=[
            pl.BlockSpec((_VB,), lambda t, vb: (t * _NVB + vb)),
            pl.BlockSpec((_VB,), lambda t, vb: (t * _NVB + vb)),
        ],
        out_shape=[
            jax.ShapeDtypeStruct((_NT * _VPAD,), jnp.float32),
            jax.ShapeDtypeStruct((_NT * _VPAD,), jnp.float32),
        ],
    )(w_all, t_t)

    offs = jnp.arange(_NT, dtype=jnp.int32) * _VPAD
    g = jnp.concatenate([feature_inputs + offs[None, :],
                         ad_feature_inputs + offs[None, :8]], axis=1)
    g = g.reshape(_NW, _RPW, _NF).transpose(0, 2, 1)

    mesh = plsc.VectorSubcoreMesh(core_axis_name="c", subcore_axis_name="s")
    out = pl.kernel(
        _sc_body,
        out_type=jax.ShapeDtypeStruct((_B,), jnp.float32),
        mesh=mesh,
        compiler_params=pltpu.CompilerParams(needs_layout_passes=False,
                                             use_tc_tiling_on_sc=False),
        scratch_types=[
            pltpu.VMEM((_NF, _RPW), jnp.int32),
            pltpu.VMEM((_NF * _RPW,), jnp.float32),
            pltpu.VMEM((_RPW,), jnp.float32),
            pltpu.VMEM((_HALF,), jnp.float32),
            pltpu.SemaphoreType.DMA,
        ],
    )(g, o_feat, o_ad, bias_vec)
    return out[:, None]


# speedup vs baseline: 8.9002x; 1.5145x over previous
import functools

import jax
import jax.numpy as jnp
from jax import lax
from jax.experimental import pallas as pl
from jax.experimental.pallas import tpu as pltpu
from jax.experimental.pallas import tpu_sc as plsc

_NUM_FIELDS = 26
_VOCAB = 100000
_VPAD = 102400
_EMB = 32
_B = 4096
_NF = 33
_NT = 25
_NC = 2
_NS = 16
_NW = _NC * _NS
_RPW = _B // _NW
_HALF = 16
_VB = 102400
_NVB = _VPAD // _VB


def _score_body(w_ref, t_ref, of_ref, oa_ref):
    s = lax.dot_general(w_ref[0], t_ref[0], (((1,), (0,)), ((), ())),
                        preferred_element_type=jnp.float32)
    of_ref[...] = s[0]
    oa_ref[...] = s[1]


def _sc_body(g_hbm, of_hbm, oa_hbm, bias_hbm, out_hbm,
             gidx_v, sbuf_v, outv_v, bias_v, sem):
    wid = lax.axis_index("s") * _NC + lax.axis_index("c")
    pltpu.sync_copy(bias_hbm, bias_v)
    pltpu.sync_copy(g_hbm.at[wid], gidx_v)
    for j in range(_NF):
        src = of_hbm if j < _NT else oa_hbm
        pltpu.async_copy(src.at[gidx_v.at[j]],
                         sbuf_v.at[pl.ds(j * _RPW, _RPW)], sem)
    pltpu.make_async_copy(of_hbm.at[pl.ds(0, _NF * _RPW)], sbuf_v, sem).wait()

    bias = bias_v[:]
    for g in range(_RPW // _HALF):
        tot = sbuf_v[pl.ds(g * _HALF, _HALF)]
        for j in range(1, _NF):
            tot = tot + sbuf_v[pl.ds(j * _RPW + g * _HALF, _HALF)]
        tot = tot + bias
        p = 1.0 / (1.0 + jnp.exp(-tot))
        outv_v[pl.ds(g * _HALF, _HALF)] = p
    pltpu.sync_copy(outv_v, out_hbm.at[pl.ds(wid * _RPW, _RPW)])


def kernel(ad_feature_inputs, feature_inputs, tables, W_meta, b_meta,
           W_pred, b_pred):
    t_t = tables.transpose(0, 2, 1)

    w0 = W_pred[0, :_EMB]
    v = W_meta.T @ w0
    w_feat = W_pred[0, _EMB:].reshape(_NT, _EMB)
    w_ad = jnp.zeros((_NT, _EMB), jnp.float32).at[:8].set(
        jnp.broadcast_to((v / _EMB)[:, None], (8, _EMB)))
    w_all = jnp.stack([w_feat, w_ad], axis=1)
    bias_c = b_pred[0] + jnp.dot(b_meta, w0)
    bias_vec = jnp.full((_HALF,), bias_c, jnp.float32)

    o_feat, o_ad = pl.pallas_call(
        _score_body,
        grid=(_NT,),
        in_specs=[
            pl.BlockSpec((1, 2, _EMB), lambda t: (t, 0, 0)),
            pl.BlockSpec((1, _EMB, _VB), lambda t: (t + 1, 0, 0)),
        ],
        out_specs=[
            pl.BlockSpec((_VB,), lambda t: (t,)),
            pl.BlockSpec((_VB,), lambda t: (t,)),
        ],
        out_shape=[
            jax.ShapeDtypeStruct((_NT * _VPAD,), jnp.float32),
            jax.ShapeDtypeStruct((_NT * _VPAD,), jnp.float32),
        ],
    )(w_all, t_t)

    offs = jnp.arange(_NT, dtype=jnp.int32) * _VPAD
    g = jnp.concatenate([feature_inputs + offs[None, :],
                         ad_feature_inputs + offs[None, :8]], axis=1)
    g = g.reshape(_NW, _RPW, _NF).transpose(0, 2, 1)

    mesh = plsc.VectorSubcoreMesh(core_axis_name="c", subcore_axis_name="s")
    out = pl.kernel(
        _sc_body,
        out_type=jax.ShapeDtypeStruct((_B,), jnp.float32),
        mesh=mesh,
        compiler_params=pltpu.CompilerParams(needs_layout_passes=False,
                                             use_tc_tiling_on_sc=False),
        scratch_types=[
            pltpu.VMEM((_NF, _RPW), jnp.int32),
            pltpu.VMEM((_NF * _RPW,), jnp.float32),
            pltpu.VMEM((_RPW,), jnp.float32),
            pltpu.VMEM((_HALF,), jnp.float32),
            pltpu.SemaphoreType.DMA,
        ],
    )(g, o_feat, o_ad, bias_vec)
    return out[:, None]
